# baseline (speedup 1.0000x reference)
import functools

import jax
import jax.numpy as jnp
from jax import lax
from jax.experimental import pallas as pl
from jax.experimental.pallas import tpu as pltpu
from jax.experimental.pallas import tpu_sc as plsc

D = 64
EPS = 1e-5
NC, NS = 2, 16
NW = NC * NS
GATHER = 128
K = 10
CHUNK = GATHER * K


def _rsqrt_newton(a):
    i = lax.bitcast_convert_type(a, jnp.int32)
    i = jnp.int32(0x5F3759DF) - lax.shift_right_arithmetic(i, 1)
    y = lax.bitcast_convert_type(i, jnp.float32)
    half_a = 0.5 * a
    for _ in range(3):
        y = y * (1.5 - half_a * y * y)
    return y


_GATHER_DNUMS = lax.GatherDimensionNumbers(
    offset_dims=(), collapsed_slice_dims=(0,), start_index_map=(0,))


def _shuffle(v, idx):
    return lax.gather(
        v, idx[:, None], _GATHER_DNUMS, slice_sizes=(1,),
        mode=lax.GatherScatterMode.PROMISE_IN_BOUNDS)


def _lane_sum(v, iota):
    for sh in (8, 4, 2, 1):
        v = v + _shuffle(v, lax.bitwise_xor(iota, jnp.int32(sh)))
    return v


def _make_sc_kernel(n_rows):
    assert n_rows % (NW * CHUNK) == 0
    per_w = n_rows // NW
    n_chunks = per_w // CHUNK
    mesh = plsc.VectorSubcoreMesh(core_axis_name="c", subcore_axis_name="s")

    @functools.partial(
        pl.kernel,
        out_type=jax.ShapeDtypeStruct((n_rows, D), jnp.float32),
        mesh=mesh,
        compiler_params=pltpu.CompilerParams(use_tc_tiling_on_sc=False),
        scratch_types=[
            pltpu.VMEM((per_w,), jnp.int32),
            pltpu.VMEM((CHUNK, D), jnp.float32),
            pltpu.VMEM((D,), jnp.float32),
            pltpu.VMEM((D,), jnp.float32),
            pltpu.SemaphoreType.DMA,
        ],
    )
    def sc_kernel(table_hbm, idx_hbm, gamma_hbm, beta_hbm, out_hbm,
                  idx_v, rows_v, gamma_v, beta_v, sem):
        wid = lax.axis_index("s") * NC + lax.axis_index("c")
        base = wid * per_w
        pltpu.sync_copy(idx_hbm.at[pl.ds(base, per_w)], idx_v)
        pltpu.sync_copy(gamma_hbm, gamma_v)
        pltpu.sync_copy(beta_hbm, beta_v)
        g = [gamma_v[pl.ds(16 * j, 16)] for j in range(4)]
        b = [beta_v[pl.ds(16 * j, 16)] for j in range(4)]
        iota = lax.iota(jnp.int32, 16)

        def process_chunk(c, _):
            off = c * CHUNK
            copies = []
            for gi in range(K):
                copies.append(pltpu.async_copy(
                    table_hbm.at[idx_v.at[pl.ds(off + gi * GATHER, GATHER)]],
                    rows_v.at[pl.ds(gi * GATHER, GATHER)],
                    sem,
                ))
            for cp in copies:
                cp.wait()

            def ln_row(r, _):
                e0 = rows_v[r, pl.ds(0, 16)]
                e1 = rows_v[r, pl.ds(16, 16)]
                e2 = rows_v[r, pl.ds(32, 16)]
                e3 = rows_v[r, pl.ds(48, 16)]
                mean = _lane_sum(e0 + e1 + e2 + e3, iota) * (1.0 / D)
                d0 = e0 - mean
                d1 = e1 - mean
                d2 = e2 - mean
                d3 = e3 - mean
                var = _lane_sum(
                    d0 * d0 + d1 * d1 + d2 * d2 + d3 * d3, iota) * (1.0 / D)
                rstd = _rsqrt_newton(var + EPS)
                rows_v[r, pl.ds(0, 16)] = d0 * (rstd * g[0]) + b[0]
                rows_v[r, pl.ds(16, 16)] = d1 * (rstd * g[1]) + b[1]
                rows_v[r, pl.ds(32, 16)] = d2 * (rstd * g[2]) + b[2]
                rows_v[r, pl.ds(48, 16)] = d3 * (rstd * g[3]) + b[3]
                return 0

            lax.fori_loop(0, CHUNK, ln_row, 0)
            pltpu.sync_copy(rows_v, out_hbm.at[pl.ds(base + off, CHUNK)])
            return 0

        lax.fori_loop(0, n_chunks, process_chunk, 0)

    return sc_kernel


def kernel(x, table, gamma, beta):
    n_rows = x.shape[0] * x.shape[1]
    idx = x.reshape(-1).astype(jnp.int32)
    out = _make_sc_kernel(n_rows)(table, idx, gamma, beta)
    return out.reshape(x.shape + (D,))
